# Initial kernel scaffold; baseline (speedup 1.0000x reference)
import functools

import jax
import jax.numpy as jnp
from jax import lax
from jax.experimental import pallas as pl
from jax.experimental.pallas import tpu as pltpu
from jax.experimental.pallas import tpu_sc as plsc

VOCAB = 1000000
EMBED_DIM = 16
BATCH = 16384
HIST = 200

NC, NS, L = 2, 16, 16
NW = NC * NS
SPW = BATCH // NW
CHUNK = 16
NCHUNK = SPW // CHUNK
IDX_PER_CHUNK = CHUNK * HIST
DMA_IDX = 128
NDMA = IDX_PER_CHUNK // DMA_IDX

_ROUND_MAGIC = 12582912.0


def _body(table_hbm, xflat_hbm, w_hbm, b_hbm, out_hbm,
          idx_v, rows_v, w_v, b_v, out_v, sem_g, sem_i):
    wid = lax.axis_index("c") * NS + lax.axis_index("s")
    base_idx = wid * (SPW * HIST)

    pltpu.sync_copy(w_hbm, w_v)
    pltpu.sync_copy(b_hbm, b_v)
    w = w_v[0, :]
    bb = b_v[0, 0]
    lanes = lax.iota(jnp.int32, L)
    zero = jnp.zeros((L,), jnp.float32)

    def chunk_body(g, carry):
        pltpu.async_copy(
            xflat_hbm.at[pl.ds(base_idx + g * IDX_PER_CHUNK, IDX_PER_CHUNK)],
            idx_v, sem_i).wait()
        cps = [
            pltpu.async_copy(
                table_hbm.at[idx_v.at[pl.ds(d * DMA_IDX, DMA_IDX)]],
                rows_v.at[pl.ds(d * DMA_IDX, DMA_IDX)],
                sem_g)
            for d in range(NDMA)
        ]
        for cp in cps:
            cp.wait()

        zv = zero
        for s in range(CHUNK):
            row0 = s * HIST

            def red(k, accs):
                a0, a1, a2, a3 = accs
                i = row0 + k * 8
                a0 = a0 + rows_v[i, :] + rows_v[i + 4, :]
                a1 = a1 + rows_v[i + 1, :] + rows_v[i + 5, :]
                a2 = a2 + rows_v[i + 2, :] + rows_v[i + 6, :]
                a3 = a3 + rows_v[i + 3, :] + rows_v[i + 7, :]
                return (a0, a1, a2, a3)

            a0, a1, a2, a3 = lax.fori_loop(
                0, HIST // 8, red, (zero, zero, zero, zero))
            acc = (a0 + a1) + (a2 + a3)
            z = jnp.sum(acc * w) * (1.0 / HIST) + bb
            zv = jnp.where(lanes == s, z, zv)

        y = 1.0 / (1.0 + jnp.exp(-zv))
        t = y * 1e4
        r = (t + _ROUND_MAGIC) - _ROUND_MAGIC
        out_v[g, :] = r * 1e-4
        return carry

    lax.fori_loop(0, NCHUNK, chunk_body, 0)
    pltpu.sync_copy(out_v, out_hbm.at[pl.ds(wid * NCHUNK, NCHUNK)])


@functools.partial(
    pl.kernel,
    out_type=jax.ShapeDtypeStruct((BATCH // L, L), jnp.float32),
    mesh=plsc.VectorSubcoreMesh(core_axis_name="c", subcore_axis_name="s"),
    scratch_types=[
        pltpu.VMEM((IDX_PER_CHUNK,), jnp.int32),
        pltpu.VMEM((IDX_PER_CHUNK, EMBED_DIM), jnp.float32),
        pltpu.VMEM((1, L), jnp.float32),
        pltpu.VMEM((1, 1), jnp.float32),
        pltpu.VMEM((NCHUNK, L), jnp.float32),
        pltpu.SemaphoreType.DMA,
        pltpu.SemaphoreType.DMA,
    ],
)
def _embedding_bag_sc(table_hbm, xflat_hbm, w_hbm, b_hbm, out_hbm,
                      idx_v, rows_v, w_v, b_v, out_v, sem_g, sem_i):
    _body(table_hbm, xflat_hbm, w_hbm, b_hbm, out_hbm,
          idx_v, rows_v, w_v, b_v, out_v, sem_g, sem_i)


@jax.jit
def kernel(x, table, W, b):
    xflat = x.reshape(-1).astype(jnp.int32)
    out = _embedding_bag_sc(table, xflat, W.reshape(1, L),
                            b.reshape(1, 1))
    return out.reshape(BATCH, 1)


# speedup vs baseline: 8.4508x; 8.4508x over previous
import functools

import jax
import jax.numpy as jnp
from jax import lax
from jax.experimental import pallas as pl
from jax.experimental.pallas import tpu as pltpu
from jax.experimental.pallas import tpu_sc as plsc

VOCAB = 1000000
EMBED_DIM = 16
BATCH = 16384
HIST = 200

NC, NS, L = 2, 16, 16
NW = NC * NS
SPW = BATCH // NW
CHUNK = 16
NCHUNK = SPW // CHUNK
IDX_PER_CHUNK = CHUNK * HIST
DMA_IDX = 128
NDMA = IDX_PER_CHUNK // DMA_IDX

_ROUND_MAGIC = 12582912.0


def _body(table_hbm, xflat_hbm, w_hbm, b_hbm, out_hbm,
          idx_v, rows_v, w_v, b_v, pooled_t, out_v, sem_g, sem_i):
    wid = lax.axis_index("c") * NS + lax.axis_index("s")
    base_idx = wid * (SPW * HIST)

    pltpu.sync_copy(w_hbm, w_v)
    pltpu.sync_copy(b_hbm, b_v)
    wrows = [w_v[d, :] for d in range(EMBED_DIM)]
    bv = b_v[0, :]
    lanes = lax.iota(jnp.int32, L)
    zero = jnp.zeros((L,), jnp.float32)

    def chunk_body(g, carry):
        pltpu.async_copy(
            xflat_hbm.at[pl.ds(base_idx + g * IDX_PER_CHUNK, IDX_PER_CHUNK)],
            idx_v, sem_i).wait()
        cps = [
            pltpu.async_copy(
                table_hbm.at[idx_v.at[pl.ds(d * DMA_IDX, DMA_IDX)]],
                rows_v.at[pl.ds(d * DMA_IDX, DMA_IDX)],
                sem_g)
            for d in range(NDMA)
        ]
        for cp in cps:
            cp.wait()

        for s in range(CHUNK):
            row0 = s * HIST

            def red(k, accs):
                a0, a1, a2, a3 = accs
                i = row0 + k * 8
                a0 = a0 + rows_v[i, :] + rows_v[i + 4, :]
                a1 = a1 + rows_v[i + 1, :] + rows_v[i + 5, :]
                a2 = a2 + rows_v[i + 2, :] + rows_v[i + 6, :]
                a3 = a3 + rows_v[i + 3, :] + rows_v[i + 7, :]
                return (a0, a1, a2, a3)

            a0, a1, a2, a3 = lax.fori_loop(
                0, HIST // 8, red, (zero, zero, zero, zero))
            acc = (a0 + a1) + (a2 + a3)
            plsc.store_scatter(
                pooled_t, [lanes, jnp.full((L,), s, jnp.int32)], acc)

        z0, z1, z2, z3 = zero, zero, zero, zero
        for d in range(0, EMBED_DIM, 4):
            z0 = z0 + pooled_t[d, :] * wrows[d]
            z1 = z1 + pooled_t[d + 1, :] * wrows[d + 1]
            z2 = z2 + pooled_t[d + 2, :] * wrows[d + 2]
            z3 = z3 + pooled_t[d + 3, :] * wrows[d + 3]
        zv = (z0 + z1) + (z2 + z3)

        zv = zv * (1.0 / HIST) + bv
        y = 1.0 / (1.0 + jnp.exp(-zv))
        t = y * 1e4
        r = (t + _ROUND_MAGIC) - _ROUND_MAGIC
        out_v[g, :] = r * 1e-4
        return carry

    lax.fori_loop(0, NCHUNK, chunk_body, 0)
    pltpu.sync_copy(out_v, out_hbm.at[pl.ds(wid * NCHUNK, NCHUNK)])


@functools.partial(
    pl.kernel,
    out_type=jax.ShapeDtypeStruct((BATCH // L, L), jnp.float32),
    mesh=plsc.VectorSubcoreMesh(core_axis_name="c", subcore_axis_name="s"),
    scratch_types=[
        pltpu.VMEM((IDX_PER_CHUNK,), jnp.int32),
        pltpu.VMEM((IDX_PER_CHUNK, EMBED_DIM), jnp.float32),
        pltpu.VMEM((EMBED_DIM, L), jnp.float32),
        pltpu.VMEM((1, L), jnp.float32),
        pltpu.VMEM((EMBED_DIM, CHUNK), jnp.float32),
        pltpu.VMEM((NCHUNK, L), jnp.float32),
        pltpu.SemaphoreType.DMA,
        pltpu.SemaphoreType.DMA,
    ],
    compiler_params=pltpu.CompilerParams(
        needs_layout_passes=False, use_tc_tiling_on_sc=False),
)
def _embedding_bag_sc(table_hbm, xflat_hbm, w_hbm, b_hbm, out_hbm,
                      idx_v, rows_v, w_v, b_v, pooled_t, out_v, sem_g, sem_i):
    _body(table_hbm, xflat_hbm, w_hbm, b_hbm, out_hbm,
          idx_v, rows_v, w_v, b_v, pooled_t, out_v, sem_g, sem_i)


@jax.jit
def kernel(x, table, W, b):
    xflat = x.reshape(-1).astype(jnp.int32)
    wb = jnp.broadcast_to(W.reshape(EMBED_DIM, 1), (EMBED_DIM, L))
    out = _embedding_bag_sc(table, xflat, wb,
                            jnp.broadcast_to(b.reshape(1, 1), (1, L)))
    return out.reshape(BATCH, 1)
